# Initial kernel scaffold; baseline (speedup 1.0000x reference)
import jax
import jax.numpy as jnp
from jax.experimental import pallas as pl


def kernel(x, mesh, W0, W1, gamma1, beta1):
    raise NotImplementedError("write your pallas kernel here")


# speedup vs baseline: 2.8475x; 2.8475x over previous
import functools

import jax
import jax.numpy as jnp
from jax import lax
from jax.experimental import pallas as pl
from jax.experimental.pallas import tpu as pltpu
from jax.experimental.pallas import tpu_sc as plsc

NC = 2
NS = 16
NW = NC * NS
CHUNK = 128
NH = 2


def _make_sc_segsum(n_nodes, n_pad_rows, c_feat, cpw):
    rpt = n_pad_rows // NS
    last_out = n_nodes - (NS - 1) * rpt
    mesh = plsc.VectorSubcoreMesh(
        core_axis_name="c", subcore_axis_name="s", num_cores=NC, num_subcores=NS
    )

    @functools.partial(
        pl.kernel,
        mesh=mesh,
        out_type=jax.ShapeDtypeStruct((NC, n_nodes, c_feat), jnp.float32),
        scratch_types=[
            pltpu.VMEM_SHARED((n_pad_rows, c_feat), jnp.float32),
            pltpu.VMEM((cpw // NH * CHUNK,), jnp.int32),
            pltpu.VMEM((cpw // NH, CHUNK), jnp.int32),
            pltpu.VMEM((CHUNK, c_feat), jnp.float32),
            pltpu.VMEM((CHUNK, c_feat), jnp.float32),
            pltpu.SemaphoreType.DMA,
            pltpu.SemaphoreType.DMA,
        ],
    )
    def segsum(table, src3, dst3, zeros, out, acc, sidx, didx, rows0, rows1,
               sem0, sem1):
        c = lax.axis_index("c")
        s = lax.axis_index("s")
        wid = c * NS + s
        hlen = cpw // NH

        z0 = s * rpt
        pltpu.sync_copy(zeros.at[pl.ds(z0, rpt)], acc.at[pl.ds(z0, rpt)])
        plsc.subcore_barrier()

        def body(i, carry):
            j0 = 2 * i
            j1 = j0 + 1
            cp0 = pltpu.async_copy(
                table.at[sidx.at[pl.ds(j0 * CHUNK, CHUNK)]], rows0, sem0
            )
            cp1 = pltpu.async_copy(
                table.at[sidx.at[pl.ds(j1 * CHUNK, CHUNK)]], rows1, sem1
            )
            cp0.wait()
            pltpu.sync_copy(rows0, acc.at[didx.at[j0]], add=True)
            cp1.wait()
            pltpu.sync_copy(rows1, acc.at[didx.at[j1]], add=True)
            return carry

        for h in range(NH):
            pltpu.sync_copy(
                src3.at[wid].at[pl.ds(h * hlen * CHUNK, hlen * CHUNK)], sidx
            )
            pltpu.sync_copy(dst3.at[wid].at[pl.ds(h * hlen, hlen)], didx)
            lax.fori_loop(0, hlen // 2, body, 0)
        plsc.subcore_barrier()

        o0 = s * rpt

        @pl.when(s < NS - 1)
        def _():
            pltpu.sync_copy(acc.at[pl.ds(o0, rpt)], out.at[c].at[pl.ds(o0, rpt)])

        @pl.when(s == NS - 1)
        def _():
            base = (NS - 1) * rpt
            pltpu.sync_copy(
                acc.at[pl.ds(base, last_out)], out.at[c].at[pl.ds(base, last_out)]
            )

    return segsum


def _tc1_body(x_ref, p_ref, w_ref, o_ref, stats_ref, acc_ref):
    i = pl.program_id(0)
    sv = x_ref[...] + p_ref[0] + p_ref[1]
    o = jnp.dot(sv, w_ref[...], preferred_element_type=jnp.float32,
                precision=lax.Precision.HIGHEST)
    o_ref[...] = o
    y = jnp.maximum(o, 0.0)

    @pl.when(i == 0)
    def _():
        acc_ref[...] = jnp.zeros_like(acc_ref)

    acc_ref[0:1] += jnp.sum(y, axis=0, keepdims=True)
    acc_ref[1:2] += jnp.sum(y * y, axis=0, keepdims=True)

    @pl.when(i == pl.num_programs(0) - 1)
    def _():
        stats_ref[...] = acc_ref[...]


def _tcbn_body(n_total, o0_ref, stats_ref, g_ref, b_ref, h_ref):
    inv_n = 1.0 / n_total
    mean = stats_ref[0:1] * inv_n
    var = stats_ref[1:2] * inv_n - mean * mean
    a = g_ref[...] * lax.rsqrt(var + 1e-5)
    bb = b_ref[...] - mean * a
    y = jnp.maximum(o0_ref[...], 0.0)
    h_ref[...] = y * a + bb


def _tc3_body(h_ref, q_ref, w_ref, o0_ref, out_ref):
    sv = h_ref[...] + q_ref[0] + q_ref[1]
    o = jnp.dot(sv, w_ref[...], preferred_element_type=jnp.float32,
                precision=lax.Precision.HIGHEST)
    out_ref[...] = jnp.maximum(o + o0_ref[...], 0.0)


def kernel(x, mesh, W0, W1, gamma1, beta1):
    n = x.shape[2]
    c_feat = x.shape[1]
    n_edges = mesh.shape[1]

    X = x[0, :, :, 0].T
    src = mesh[0].astype(jnp.int32)
    dst = mesh[1].astype(jnp.int32)

    cpw = -(-n_edges // (NW * CHUNK))
    cpw = -(-cpw // (2 * NH)) * (2 * NH)
    e_pad = NW * cpw * CHUNK
    pad = e_pad - n_edges
    src_p = jnp.concatenate([src, jnp.zeros((pad,), jnp.int32)])
    dst_p = jnp.concatenate([dst, jnp.full((pad,), n, jnp.int32)])
    src3 = src_p.reshape(NW, cpw * CHUNK)
    dst3 = dst_p.reshape(NW, cpw, CHUNK)

    n_pad_rows = -(-(n + 1) // (NS * 8)) * (NS * 8)
    zeros = jnp.zeros((n_pad_rows, c_feat), jnp.float32)

    segsum = _make_sc_segsum(n, n_pad_rows, c_feat, cpw)

    bn = 1000
    grid = (n // bn,)
    blk = lambda i: (i, 0)
    p_spec = pl.BlockSpec((NC, bn, c_feat), lambda i: (0, i, 0))
    w_spec = pl.BlockSpec((c_feat, c_feat), lambda i: (0, 0))
    full_spec = pl.BlockSpec((bn, c_feat), blk)

    P = segsum(X, src3, dst3, zeros)
    out0, stats = pl.pallas_call(
        _tc1_body,
        grid=grid,
        in_specs=[full_spec, p_spec, w_spec],
        out_specs=[full_spec, pl.BlockSpec((2, c_feat), lambda i: (0, 0))],
        out_shape=[
            jax.ShapeDtypeStruct((n, c_feat), jnp.float32),
            jax.ShapeDtypeStruct((2, c_feat), jnp.float32),
        ],
        scratch_shapes=[pltpu.VMEM((2, c_feat), jnp.float32)],
    )(X, P, W0.T)

    H = pl.pallas_call(
        functools.partial(_tcbn_body, float(n)),
        grid=grid,
        in_specs=[
            full_spec,
            pl.BlockSpec((2, c_feat), lambda i: (0, 0)),
            pl.BlockSpec((1, c_feat), lambda i: (0, 0)),
            pl.BlockSpec((1, c_feat), lambda i: (0, 0)),
        ],
        out_specs=full_spec,
        out_shape=jax.ShapeDtypeStruct((n, c_feat), jnp.float32),
    )(out0, stats, gamma1.reshape(1, -1), beta1.reshape(1, -1))

    Q = segsum(H, src3, dst3, zeros)
    F = pl.pallas_call(
        _tc3_body,
        grid=grid,
        in_specs=[full_spec, p_spec, w_spec, full_spec],
        out_specs=full_spec,
        out_shape=jax.ShapeDtypeStruct((n, c_feat), jnp.float32),
    )(H, Q, W1.T, out0)

    return F.T[None, :, :, None]


# speedup vs baseline: 3.4172x; 1.2000x over previous
import functools

import jax
import jax.numpy as jnp
from jax import lax
from jax.experimental import pallas as pl
from jax.experimental.pallas import tpu as pltpu
from jax.experimental.pallas import tpu_sc as plsc

NC = 2
NS = 16
NW = NC * NS
CHUNK = 128
WIN = 8


def _make_sc_segsum(n_nodes, n_pad_rows, c_feat, ca, cb):
    rpt = n_pad_rows // NS
    last_out = n_nodes - (NS - 1) * rpt
    mesh = plsc.VectorSubcoreMesh(
        core_axis_name="c", subcore_axis_name="s", num_cores=NC, num_subcores=NS
    )

    @functools.partial(
        pl.kernel,
        mesh=mesh,
        out_type=jax.ShapeDtypeStruct((NC, n_nodes, c_feat), jnp.float32),
        scratch_types=[
            pltpu.VMEM_SHARED((n_pad_rows, c_feat), jnp.float32),
            pltpu.VMEM((WIN * CHUNK,), jnp.int32),
            pltpu.VMEM((WIN, CHUNK), jnp.int32),
            pltpu.VMEM((CHUNK, c_feat), jnp.float32),
            pltpu.VMEM((CHUNK, c_feat), jnp.float32),
            pltpu.SemaphoreType.DMA,
            pltpu.SemaphoreType.DMA,
        ],
    )
    def segsum(table, src2, dst2, zeros, out, acc, sidx, didx, rows0, rows1,
               sem0, sem1):
        c = lax.axis_index("c")
        s = lax.axis_index("s")
        chunk_base = jnp.where(c == 0, s * ca, NS * ca + s * cb)
        n_win = jnp.where(c == 0, ca // WIN, cb // WIN)

        z0 = s * rpt
        pltpu.sync_copy(zeros.at[pl.ds(z0, rpt)], acc.at[pl.ds(z0, rpt)])
        plsc.subcore_barrier()

        def window(t, carry):
            wc = chunk_base + t * WIN
            pltpu.sync_copy(src2.at[pl.ds(wc * CHUNK, WIN * CHUNK)], sidx)
            pltpu.sync_copy(dst2.at[pl.ds(wc, WIN)], didx)
            for p in range(WIN // 2):
                j0 = 2 * p
                j1 = j0 + 1
                cp0 = pltpu.async_copy(
                    table.at[sidx.at[pl.ds(j0 * CHUNK, CHUNK)]], rows0, sem0
                )
                cp1 = pltpu.async_copy(
                    table.at[sidx.at[pl.ds(j1 * CHUNK, CHUNK)]], rows1, sem1
                )
                cp0.wait()
                pltpu.sync_copy(rows0, acc.at[didx.at[j0]], add=True)
                cp1.wait()
                pltpu.sync_copy(rows1, acc.at[didx.at[j1]], add=True)
            return carry

        lax.fori_loop(0, n_win, window, 0)
        plsc.subcore_barrier()

        o0 = s * rpt

        @pl.when(s < NS - 1)
        def _():
            pltpu.sync_copy(acc.at[pl.ds(o0, rpt)], out.at[c].at[pl.ds(o0, rpt)])

        @pl.when(s == NS - 1)
        def _():
            base = (NS - 1) * rpt
            pltpu.sync_copy(
                acc.at[pl.ds(base, last_out)], out.at[c].at[pl.ds(base, last_out)]
            )

    return segsum


def _tc1_body(x_ref, p_ref, w_ref, o_ref, stats_ref, acc_ref):
    i = pl.program_id(0)
    sv = x_ref[...] + p_ref[0] + p_ref[1]
    o = jnp.dot(sv, w_ref[...], preferred_element_type=jnp.float32,
                precision=lax.Precision.HIGHEST)
    o_ref[...] = o
    y = jnp.maximum(o, 0.0)

    @pl.when(i == 0)
    def _():
        acc_ref[...] = jnp.zeros_like(acc_ref)

    acc_ref[0:1] += jnp.sum(y, axis=0, keepdims=True)
    acc_ref[1:2] += jnp.sum(y * y, axis=0, keepdims=True)

    @pl.when(i == pl.num_programs(0) - 1)
    def _():
        stats_ref[...] = acc_ref[...]


def _tcbn_body(n_total, o0_ref, stats_ref, g_ref, b_ref, h_ref):
    inv_n = 1.0 / n_total
    mean = stats_ref[0:1] * inv_n
    var = stats_ref[1:2] * inv_n - mean * mean
    a = g_ref[...] * lax.rsqrt(var + 1e-5)
    bb = b_ref[...] - mean * a
    y = jnp.maximum(o0_ref[...], 0.0)
    h_ref[...] = y * a + bb


def _tc3_body(h_ref, q_ref, w_ref, o0_ref, out_ref):
    sv = h_ref[...] + q_ref[0] + q_ref[1]
    o = jnp.dot(sv, w_ref[...], preferred_element_type=jnp.float32,
                precision=lax.Precision.HIGHEST)
    out_ref[...] = jnp.maximum(o + o0_ref[...], 0.0)


def kernel(x, mesh, W0, W1, gamma1, beta1):
    n = x.shape[2]
    c_feat = x.shape[1]
    n_edges = mesh.shape[1]

    X = x[0, :, :, 0].T
    src = mesh[0].astype(jnp.int32)
    dst = mesh[1].astype(jnp.int32)

    tot = -(-n_edges // (NS * CHUNK))
    ca = -(-int(tot * 0.80) // WIN) * WIN
    cb = max(-(-(tot - ca) // WIN) * WIN, WIN)
    e_pad = NS * (ca + cb) * CHUNK
    pad = e_pad - n_edges
    src_p = jnp.concatenate([src, jnp.zeros((pad,), jnp.int32)])
    dst_p = jnp.concatenate([dst, jnp.full((pad,), n, jnp.int32)])
    dst2 = dst_p.reshape(e_pad // CHUNK, CHUNK)

    n_pad_rows = -(-(n + 1) // (NS * 8)) * (NS * 8)
    zeros = jnp.zeros((n_pad_rows, c_feat), jnp.float32)

    segsum = _make_sc_segsum(n, n_pad_rows, c_feat, ca, cb)

    bn = 1000
    grid = (n // bn,)
    blk = lambda i: (i, 0)
    p_spec = pl.BlockSpec((NC, bn, c_feat), lambda i: (0, i, 0))
    w_spec = pl.BlockSpec((c_feat, c_feat), lambda i: (0, 0))
    full_spec = pl.BlockSpec((bn, c_feat), blk)

    P = segsum(X, src_p, dst2, zeros)
    out0, stats = pl.pallas_call(
        _tc1_body,
        grid=grid,
        in_specs=[full_spec, p_spec, w_spec],
        out_specs=[full_spec, pl.BlockSpec((2, c_feat), lambda i: (0, 0))],
        out_shape=[
            jax.ShapeDtypeStruct((n, c_feat), jnp.float32),
            jax.ShapeDtypeStruct((2, c_feat), jnp.float32),
        ],
        scratch_shapes=[pltpu.VMEM((2, c_feat), jnp.float32)],
    )(X, P, W0.T)

    H = pl.pallas_call(
        functools.partial(_tcbn_body, float(n)),
        grid=grid,
        in_specs=[
            full_spec,
            pl.BlockSpec((2, c_feat), lambda i: (0, 0)),
            pl.BlockSpec((1, c_feat), lambda i: (0, 0)),
            pl.BlockSpec((1, c_feat), lambda i: (0, 0)),
        ],
        out_specs=full_spec,
        out_shape=jax.ShapeDtypeStruct((n, c_feat), jnp.float32),
    )(out0, stats, gamma1.reshape(1, -1), beta1.reshape(1, -1))

    Q = segsum(H, src_p, dst2, zeros)
    F = pl.pallas_call(
        _tc3_body,
        grid=grid,
        in_specs=[full_spec, p_spec, w_spec, full_spec],
        out_specs=full_spec,
        out_shape=jax.ShapeDtypeStruct((n, c_feat), jnp.float32),
    )(H, Q, W1.T, out0)

    return F.T[None, :, :, None]
